# Initial kernel scaffold; baseline (speedup 1.0000x reference)
import jax
import jax.numpy as jnp
from jax.experimental import pallas as pl


def kernel(feature, xyz, neigh_idx, W1, g1, b1, Wb1, gb1, bb1, Wf1, Wm1, gm1, bm1, Wb2, gb2, bb2, Wf2, Wm2, gm2, bm2, W2, g2, b2, Ws, gs, bs):
    raise NotImplementedError("write your pallas kernel here")


# speedup vs baseline: 14.1498x; 14.1498x over previous
import functools

import jax
import jax.numpy as jnp
from jax import lax
from jax.experimental import pallas as pl
from jax.experimental.pallas import tpu as pltpu
from jax.experimental.pallas import tpu_sc as plsc

B, N, K = 2, 10000, 16
D_IN, D_OUT = 128, 128
D2 = D_OUT // 2
TW = 128
TN = 400
NT = N // TN
TNK = TN * K
ROWS = B * N * K

NC, NS = 2, 16
NW = NC * NS
RW = ROWS // NW
CH = 80
NCH = RW // CH


def _sc_gather_body(tab_hbm, nidx_hbm, out_hbm, idx_v, rows_v, sem):
    wid = lax.axis_index("s") * NC + lax.axis_index("c")
    base = wid * RW
    boff = (wid // (NW // B)) * N

    def chunk(j, carry):
        st = base + j * CH
        pltpu.sync_copy(nidx_hbm.at[pl.ds(st, CH)], idx_v)
        for i in range(CH // 16):
            idx_v[pl.ds(i * 16, 16)] = idx_v[pl.ds(i * 16, 16)] + boff
        pltpu.async_copy(tab_hbm.at[idx_v], rows_v, sem).wait()
        pltpu.sync_copy(rows_v, out_hbm.at[pl.ds(st, CH)])
        return carry

    lax.fori_loop(0, NCH, chunk, 0)


def _make_sc_gather():
    mesh = plsc.VectorSubcoreMesh(core_axis_name="c", subcore_axis_name="s")
    return pl.kernel(
        _sc_gather_body,
        out_type=jax.ShapeDtypeStruct((ROWS, TW), jnp.float32),
        mesh=mesh,
        scratch_types=[
            pltpu.VMEM((CH,), jnp.int32),
            pltpu.VMEM((CH, TW), jnp.float32),
            pltpu.SemaphoreType.DMA,
        ],
    )


def _stage_a_body(f_ref, xyz_ref, w1t_ref, g1_ref, b1_ref, out_ref):
    f = f_ref[0]
    pc1 = jnp.dot(f, w1t_ref[...], preferred_element_type=jnp.float32)
    pc1 = jnp.maximum(pc1 * g1_ref[...] + b1_ref[...], 0.0)
    out_ref[:, 0:64] = pc1
    out_ref[:, 64:67] = xyz_ref[0]


def _rel_pos_feat(g_rows, xyz_t, wb1t_ref, gb1_ref, bb1_ref):
    nxyz = g_rows[:, 64:67]
    xt = jnp.broadcast_to(xyz_t[:, None, :], (TN, K, 3)).reshape(TNK, 3)
    rel = xt - nxyz
    dis2 = jnp.sum(rel * rel, axis=1, keepdims=True)
    dis = jnp.sqrt(jnp.maximum(dis2, 1e-20))
    wb1t = wb1t_ref[...]
    pre = (dis * wb1t[0:1, :]
           + lax.dot_general(rel, wb1t[1:4, :], (((1,), (0,)), ((), ())),
                             preferred_element_type=jnp.float32)
           + lax.dot_general(xt, wb1t[4:7, :], (((1,), (0,)), ((), ())),
                             preferred_element_type=jnp.float32)
           + lax.dot_general(nxyz, wb1t[7:10, :], (((1,), (0,)), ((), ())),
                             preferred_element_type=jnp.float32))
    return jnp.maximum(pre * gb1_ref[...] + bb1_ref[...], 0.0)


def _att_pool_rows(cat, wft_ref, wmt_ref, gm_ref, bm_ref):
    c = cat.shape[1]
    att = jnp.dot(cat, wft_ref[...], preferred_element_type=jnp.float32)
    att3 = att.reshape(TN, K, c)
    m = jnp.max(att3, axis=1, keepdims=True)
    e = jnp.exp(att3 - m)
    s = e / jnp.sum(e, axis=1, keepdims=True)
    f = jnp.sum(cat.reshape(TN, K, c) * s, axis=1)
    agg = jnp.dot(f, wmt_ref[...], preferred_element_type=jnp.float32)
    return jnp.maximum(agg * gm_ref[...] + bm_ref[...], 0.0)


def _stage_c_body(g1_ref, xyz_ref, wb1t_ref, gb1_ref, bb1_ref,
                  wf1t_ref, wm1t_ref, gm1_ref, bm1_ref, out_ref):
    g1 = g1_ref[...]
    xyz_t = xyz_ref[0]
    f_xyz1 = _rel_pos_feat(g1, xyz_t, wb1t_ref, gb1_ref, bb1_ref)
    cat = jnp.concatenate([g1[:, 0:64], f_xyz1], axis=1)
    agg1 = _att_pool_rows(cat, wf1t_ref, wm1t_ref, gm1_ref, bm1_ref)
    out_ref[:, 0:64] = agg1
    out_ref[:, 64:67] = xyz_t


def _stage_e_body(g2_ref, xyz_ref, f_ref,
                  wb1t_ref, gb1_ref, bb1_ref, wb2t_ref, gb2_ref, bb2_ref,
                  wf2t_ref, wm2t_ref, gm2_ref, bm2_ref,
                  w2t_ref, g2_ref2, b2_ref, wst_ref, gs_ref, bs_ref, out_ref):
    g2 = g2_ref[...]
    xyz_t = xyz_ref[0]
    f_xyz1 = _rel_pos_feat(g2, xyz_t, wb1t_ref, gb1_ref, bb1_ref)
    f_xyz2 = jnp.dot(f_xyz1, wb2t_ref[...], preferred_element_type=jnp.float32)
    f_xyz2 = jnp.maximum(f_xyz2 * gb2_ref[...] + bb2_ref[...], 0.0)
    cat = jnp.concatenate([g2[:, 0:64], f_xyz2], axis=1)
    agg2 = _att_pool_rows(cat, wf2t_ref, wm2t_ref, gm2_ref, bm2_ref)
    f_out = jnp.dot(agg2, w2t_ref[...], preferred_element_type=jnp.float32)
    f_out = f_out * g2_ref2[...] + b2_ref[...]
    ft = f_ref[0]
    sc = jnp.dot(ft, wst_ref[...], preferred_element_type=jnp.float32)
    sc = sc * gs_ref[...] + bs_ref[...]
    o = f_out + sc
    o = jnp.where(o >= 0, o, 0.2 * o)
    out_ref[...] = o.reshape(1, TN, 2 * D_OUT)


def _full_spec(shape):
    return pl.BlockSpec(shape, lambda b, t: tuple(0 for _ in shape))


def kernel(feature, xyz, neigh_idx, W1, g1, b1, Wb1, gb1, bb1, Wf1, Wm1, gm1,
           bm1, Wb2, gb2, bb2, Wf2, Wm2, gm2, bm2, W2, g2, b2, Ws, gs, bs):
    nidx_flat = neigh_idx.astype(jnp.int32).reshape(ROWS)
    featr = jnp.transpose(feature[..., 0], (0, 2, 1))
    row = lambda v: v.reshape(1, -1)

    t1 = pl.pallas_call(
        _stage_a_body,
        grid=(B, NT),
        in_specs=[
            pl.BlockSpec((1, TN, D_IN), lambda b, t: (b, t, 0)),
            pl.BlockSpec((1, TN, 3), lambda b, t: (b, t, 0)),
            _full_spec((D_IN, D2)),
            _full_spec((1, D2)),
            _full_spec((1, D2)),
        ],
        out_specs=pl.BlockSpec((TN, TW), lambda b, t: (b * NT + t, 0)),
        out_shape=jax.ShapeDtypeStruct((B * N, TW), jnp.float32),
    )(featr, xyz, W1.T, row(g1), row(b1))

    sc_gather = _make_sc_gather()
    g1rows = sc_gather(t1, nidx_flat)

    t2 = pl.pallas_call(
        _stage_c_body,
        grid=(B, NT),
        in_specs=[
            pl.BlockSpec((TNK, TW), lambda b, t: (b * NT + t, 0)),
            pl.BlockSpec((1, TN, 3), lambda b, t: (b, t, 0)),
            _full_spec((10, D2)),
            _full_spec((1, D2)),
            _full_spec((1, D2)),
            _full_spec((D_OUT, D_OUT)),
            _full_spec((D_OUT, D2)),
            _full_spec((1, D2)),
            _full_spec((1, D2)),
        ],
        out_specs=pl.BlockSpec((TN, TW), lambda b, t: (b * NT + t, 0)),
        out_shape=jax.ShapeDtypeStruct((B * N, TW), jnp.float32),
    )(g1rows, xyz, Wb1.T, row(gb1), row(bb1), Wf1.T, Wm1.T, row(gm1), row(bm1))

    g2rows = sc_gather(t2, nidx_flat)

    out = pl.pallas_call(
        _stage_e_body,
        grid=(B, NT),
        in_specs=[
            pl.BlockSpec((TNK, TW), lambda b, t: (b * NT + t, 0)),
            pl.BlockSpec((1, TN, 3), lambda b, t: (b, t, 0)),
            pl.BlockSpec((1, TN, D_IN), lambda b, t: (b, t, 0)),
            _full_spec((10, D2)),
            _full_spec((1, D2)),
            _full_spec((1, D2)),
            _full_spec((D2, D2)),
            _full_spec((1, D2)),
            _full_spec((1, D2)),
            _full_spec((D_OUT, D_OUT)),
            _full_spec((D_OUT, D_OUT)),
            _full_spec((1, D_OUT)),
            _full_spec((1, D_OUT)),
            _full_spec((D_OUT, 2 * D_OUT)),
            _full_spec((1, 2 * D_OUT)),
            _full_spec((1, 2 * D_OUT)),
            _full_spec((D_IN, 2 * D_OUT)),
            _full_spec((1, 2 * D_OUT)),
            _full_spec((1, 2 * D_OUT)),
        ],
        out_specs=pl.BlockSpec((1, TN, 2 * D_OUT), lambda b, t: (b, t, 0)),
        out_shape=jax.ShapeDtypeStruct((B, N, 2 * D_OUT), jnp.float32),
    )(g2rows, xyz, featr, Wb1.T, row(gb1), row(bb1), Wb2.T, row(gb2),
      row(bb2), Wf2.T, Wm2.T, row(gm2), row(bm2), W2.T, row(g2), row(b2),
      Ws.T, row(gs), row(bs))
    return jnp.transpose(out, (0, 2, 1))[..., None]


# speedup vs baseline: 18.7747x; 1.3269x over previous
import functools

import jax
import jax.numpy as jnp
from jax import lax
from jax.experimental import pallas as pl
from jax.experimental.pallas import tpu as pltpu
from jax.experimental.pallas import tpu_sc as plsc

B, N, K = 2, 10000, 16
D_IN, D_OUT = 128, 128
D2 = D_OUT // 2
TW = 128
TN = 400
NT = N // TN
TNK = TN * K
ROWS = B * N * K

NC, NS = 2, 16
NW = NC * NS
RW = ROWS // NW
CH = 80
NCH = RW // CH


def _sc_gather_body(tab_hbm, nidx_hbm, out_hbm, idx_v, rows0, rows1,
                    gsem0, gsem1, wsem0, wsem1):
    wid = lax.axis_index("s") * NC + lax.axis_index("c")
    base = wid * RW
    boff = (wid // (NW // B)) * N

    pltpu.sync_copy(nidx_hbm.at[pl.ds(base, RW)], idx_v)

    def addoff(i, carry):
        idx_v[pl.ds(i * 16, 16)] = idx_v[pl.ds(i * 16, 16)] + boff
        return carry

    lax.fori_loop(0, RW // 16, addoff, 0)

    bufs = ((rows0, gsem0, wsem0), (rows1, gsem1, wsem1))

    def do_chunk(j, rows_v, gsem, wsem):
        @pl.when(j >= 2)
        def _():
            pltpu.make_async_copy(rows_v, out_hbm.at[pl.ds(base, CH)],
                                  wsem).wait()
        gcp = pltpu.make_async_copy(tab_hbm.at[idx_v.at[pl.ds(j * CH, CH)]],
                                    rows_v, gsem)
        gcp.start()
        gcp.wait()
        pltpu.make_async_copy(rows_v, out_hbm.at[pl.ds(base + j * CH, CH)],
                              wsem).start()

    def pair(i2, carry):
        for b2 in range(2):
            rows_v, gsem, wsem = bufs[b2]
            do_chunk(2 * i2 + b2, rows_v, gsem, wsem)
        return carry

    lax.fori_loop(0, NCH // 2, pair, 0)
    if NCH % 2:
        do_chunk(NCH - 1, *bufs[0])
        pltpu.make_async_copy(rows1, out_hbm.at[pl.ds(base, CH)], wsem1).wait()
        pltpu.make_async_copy(rows0, out_hbm.at[pl.ds(base, CH)], wsem0).wait()
    else:
        pltpu.make_async_copy(rows0, out_hbm.at[pl.ds(base, CH)], wsem0).wait()
        pltpu.make_async_copy(rows1, out_hbm.at[pl.ds(base, CH)], wsem1).wait()


def _make_sc_gather():
    mesh = plsc.VectorSubcoreMesh(core_axis_name="c", subcore_axis_name="s")
    return pl.kernel(
        _sc_gather_body,
        out_type=jax.ShapeDtypeStruct((ROWS, TW), jnp.float32),
        mesh=mesh,
        scratch_types=[
            pltpu.VMEM((RW,), jnp.int32),
            pltpu.VMEM((CH, TW), jnp.float32),
            pltpu.VMEM((CH, TW), jnp.float32),
            pltpu.SemaphoreType.DMA,
            pltpu.SemaphoreType.DMA,
            pltpu.SemaphoreType.DMA,
            pltpu.SemaphoreType.DMA,
        ],
    )


def _stage_a_body(f_ref, xyz_ref, w1t_ref, g1_ref, b1_ref, out_ref):
    f = f_ref[0]
    pc1 = jnp.dot(f, w1t_ref[...], preferred_element_type=jnp.float32)
    pc1 = jnp.maximum(pc1 * g1_ref[...] + b1_ref[...], 0.0)
    out_ref[:, 0:64] = pc1
    out_ref[:, 64:67] = xyz_ref[0]


def _rel_pos_feat(nxyz, xyz_t, w0c_ref, wxt_ref, wn_ref, gb1_ref, bb1_ref):
    rel3 = xyz_t[:, None, :] - nxyz.reshape(TN, K, 3)
    rel = rel3.reshape(TNK, 3)
    relt = jnp.transpose(rel)
    dis2t = jnp.sum(relt * relt, axis=0, keepdims=True)
    dist = jnp.sqrt(jnp.maximum(dis2t, 1e-20))
    dis_t64 = jnp.dot(w0c_ref[...], dist,
                      preferred_element_type=jnp.float32)
    dis_term = jnp.transpose(dis_t64)
    xta = jnp.dot(xyz_t, wxt_ref[...], preferred_element_type=jnp.float32)
    pre3 = (dis_term
            + lax.dot_general(nxyz, wn_ref[...], (((1,), (0,)), ((), ())),
                              preferred_element_type=jnp.float32)
            ).reshape(TN, K, D2) + xta[:, None, :]
    pre = pre3.reshape(TNK, D2)
    return jnp.maximum(pre * gb1_ref[...] + bb1_ref[...], 0.0)


def _att_pool_rows(cat, wft_ref, wmt_ref, gm_ref, bm_ref):
    c = cat.shape[1]
    att = jnp.dot(cat, wft_ref[...], preferred_element_type=jnp.float32)
    e = jnp.exp(att.reshape(TN, K, c))
    num = jnp.sum(cat.reshape(TN, K, c) * e, axis=1)
    den = jnp.sum(e, axis=1)
    f = num / den
    agg = jnp.dot(f, wmt_ref[...], preferred_element_type=jnp.float32)
    return jnp.maximum(agg * gm_ref[...] + bm_ref[...], 0.0)


def _stage_c_body(g1_ref, xyz_ref, w0c_ref, wxt_ref, wn_ref, gb1_ref, bb1_ref,
                  wf1t_ref, wm1t_ref, gm1_ref, bm1_ref, out_ref):
    xyz_t = xyz_ref[0]
    f_xyz1 = _rel_pos_feat(g1_ref[:, 64:67], xyz_t, w0c_ref, wxt_ref, wn_ref,
                           gb1_ref, bb1_ref)
    cat = jnp.concatenate([g1_ref[:, 0:64], f_xyz1], axis=1)
    agg1 = _att_pool_rows(cat, wf1t_ref, wm1t_ref, gm1_ref, bm1_ref)
    out_ref[:, 0:64] = agg1
    out_ref[:, 64:67] = xyz_t


def _stage_e_body(g2_ref, xyz_ref, f_ref,
                  w0c_ref, wxt_ref, wn_ref, gb1_ref, bb1_ref,
                  wb2t_ref, gb2_ref, bb2_ref,
                  wf2t_ref, wm2t_ref, gm2_ref, bm2_ref,
                  w2t_ref, g2_ref2, b2_ref, wst_ref, gs_ref, bs_ref, out_ref):
    xyz_t = xyz_ref[0]
    f_xyz1 = _rel_pos_feat(g2_ref[:, 64:67], xyz_t, w0c_ref, wxt_ref, wn_ref,
                           gb1_ref, bb1_ref)
    f_xyz2 = jnp.dot(f_xyz1, wb2t_ref[...], preferred_element_type=jnp.float32)
    f_xyz2 = jnp.maximum(f_xyz2 * gb2_ref[...] + bb2_ref[...], 0.0)
    cat = jnp.concatenate([g2_ref[:, 0:64], f_xyz2], axis=1)
    agg2 = _att_pool_rows(cat, wf2t_ref, wm2t_ref, gm2_ref, bm2_ref)
    f_out = jnp.dot(agg2, w2t_ref[...], preferred_element_type=jnp.float32)
    f_out = f_out * g2_ref2[...] + b2_ref[...]
    ft = f_ref[0]
    sc = jnp.dot(ft, wst_ref[...], preferred_element_type=jnp.float32)
    sc = sc * gs_ref[...] + bs_ref[...]
    o = f_out + sc
    o = jnp.where(o >= 0, o, 0.2 * o)
    out_ref[...] = o.reshape(1, TN, 2 * D_OUT)


def _full_spec(shape):
    return pl.BlockSpec(shape, lambda b, t: tuple(0 for _ in shape))


def kernel(feature, xyz, neigh_idx, W1, g1, b1, Wb1, gb1, bb1, Wf1, Wm1, gm1,
           bm1, Wb2, gb2, bb2, Wf2, Wm2, gm2, bm2, W2, g2, b2, Ws, gs, bs):
    nidx_flat = neigh_idx.astype(jnp.int32).reshape(ROWS)
    featr = jnp.transpose(feature[..., 0], (0, 2, 1))
    row = lambda v: v.reshape(1, -1)
    w0c = Wb1[:, 0:1]
    wxt = (Wb1[:, 1:4] + Wb1[:, 4:7]).T
    wn = (Wb1[:, 7:10] - Wb1[:, 1:4]).T

    t1 = pl.pallas_call(
        _stage_a_body,
        grid=(B, NT),
        in_specs=[
            pl.BlockSpec((1, TN, D_IN), lambda b, t: (b, t, 0)),
            pl.BlockSpec((1, TN, 3), lambda b, t: (b, t, 0)),
            _full_spec((D_IN, D2)),
            _full_spec((1, D2)),
            _full_spec((1, D2)),
        ],
        out_specs=pl.BlockSpec((TN, TW), lambda b, t: (b * NT + t, 0)),
        out_shape=jax.ShapeDtypeStruct((B * N, TW), jnp.float32),
    )(featr, xyz, W1.T, row(g1), row(b1))

    sc_gather = _make_sc_gather()
    g1rows = sc_gather(t1, nidx_flat)

    t2 = pl.pallas_call(
        _stage_c_body,
        grid=(B, NT),
        in_specs=[
            pl.BlockSpec((TNK, TW), lambda b, t: (b * NT + t, 0)),
            pl.BlockSpec((1, TN, 3), lambda b, t: (b, t, 0)),
            _full_spec((D2, 1)),
            _full_spec((3, D2)),
            _full_spec((3, D2)),
            _full_spec((1, D2)),
            _full_spec((1, D2)),
            _full_spec((D_OUT, D_OUT)),
            _full_spec((D_OUT, D2)),
            _full_spec((1, D2)),
            _full_spec((1, D2)),
        ],
        out_specs=pl.BlockSpec((TN, TW), lambda b, t: (b * NT + t, 0)),
        out_shape=jax.ShapeDtypeStruct((B * N, TW), jnp.float32),
    )(g1rows, xyz, w0c, wxt, wn, row(gb1), row(bb1), Wf1.T, Wm1.T, row(gm1),
      row(bm1))

    g2rows = sc_gather(t2, nidx_flat)

    out = pl.pallas_call(
        _stage_e_body,
        grid=(B, NT),
        in_specs=[
            pl.BlockSpec((TNK, TW), lambda b, t: (b * NT + t, 0)),
            pl.BlockSpec((1, TN, 3), lambda b, t: (b, t, 0)),
            pl.BlockSpec((1, TN, D_IN), lambda b, t: (b, t, 0)),
            _full_spec((D2, 1)),
            _full_spec((3, D2)),
            _full_spec((3, D2)),
            _full_spec((1, D2)),
            _full_spec((1, D2)),
            _full_spec((D2, D2)),
            _full_spec((1, D2)),
            _full_spec((1, D2)),
            _full_spec((D_OUT, D_OUT)),
            _full_spec((D_OUT, D_OUT)),
            _full_spec((1, D_OUT)),
            _full_spec((1, D_OUT)),
            _full_spec((D_OUT, 2 * D_OUT)),
            _full_spec((1, 2 * D_OUT)),
            _full_spec((1, 2 * D_OUT)),
            _full_spec((D_IN, 2 * D_OUT)),
            _full_spec((1, 2 * D_OUT)),
            _full_spec((1, 2 * D_OUT)),
        ],
        out_specs=pl.BlockSpec((1, TN, 2 * D_OUT), lambda b, t: (b, t, 0)),
        out_shape=jax.ShapeDtypeStruct((B, N, 2 * D_OUT), jnp.float32),
    )(g2rows, xyz, featr, w0c, wxt, wn, row(gb1), row(bb1), Wb2.T, row(gb2),
      row(bb2), Wf2.T, Wm2.T, row(gm2), row(bm2), W2.T, row(g2), row(b2),
      Ws.T, row(gs), row(bs))
    return jnp.transpose(out, (0, 2, 1))[..., None]
